# baseline (speedup 1.0000x reference)
import jax
import jax.numpy as jnp
from jax.experimental import pallas as pl
from jax.experimental.pallas import tpu as pltpu

_B = 2
_N = 20000
_R = 160
_C = 128
_K = 6000
_MAX_OUT = 1000
_TH = 0.7
_STD = (0.1, 0.1, 0.2, 0.2)
_ONE_BITS = 0x3F800000


def _nms_body(s_ref, d0_ref, d1_ref, d2_ref, d3_ref,
              a0_ref, a1_ref, a2_ref, a3_ref, out_ref, *scr):
    SC = [scr[0], scr[5]]
    Y1 = [scr[1], scr[6]]
    X1 = [scr[2], scr[7]]
    Y2 = [scr[3], scr[8]]
    X2 = [scr[4], scr[9]]

    ay1, ax1, ay2, ax2 = a0_ref[...], a1_ref[...], a2_ref[...], a3_ref[...]
    ah = ay2 - ay1
    aw = ax2 - ax1
    acy = ay1 + 0.5 * ah
    acx = ax1 + 0.5 * aw

    li = jax.lax.broadcasted_iota(jnp.int32, (_C, _C), 0)
    lj = jax.lax.broadcasted_iota(jnp.int32, (_C, _C), 1)
    tri_incl = (li <= lj).astype(jnp.float32)
    ri = jax.lax.broadcasted_iota(jnp.int32, (_R, _R), 0)
    rj = jax.lax.broadcasted_iota(jnp.int32, (_R, _R), 1)
    tri_strict = (rj < ri).astype(jnp.float32)

    areas = []
    for b in range(_B):
        s = s_ref[b]
        cy = acy + (d0_ref[b] * _STD[0]) * ah
        cx = acx + (d1_ref[b] * _STD[1]) * aw
        h = ah * jnp.exp(d2_ref[b] * _STD[2])
        w = aw * jnp.exp(d3_ref[b] * _STD[3])
        y1 = jnp.clip(cy - 0.5 * h, 0.0, 1.0)
        x1 = jnp.clip(cx - 0.5 * w, 0.0, 1.0)
        y2 = jnp.clip((cy - 0.5 * h) + h, 0.0, 1.0)
        x2 = jnp.clip((cx - 0.5 * w) + w, 0.0, 1.0)
        areas.append((y2 - y1) * (x2 - x1))
        Y1[b][...] = y1
        X1[b][...] = x1
        Y2[b][...] = y2
        X2[b][...] = x2

        bits = jax.lax.bitcast_convert_type(s, jnp.int32)

        def _bis(_, lh):
            lo, hi = lh
            mid = (lo + hi) // 2
            cge = jnp.sum((bits >= mid).astype(jnp.float32))
            take = cge >= float(_K)
            return (jnp.where(take, mid, lo), jnp.where(take, hi, mid))

        lo, _ = jax.lax.fori_loop(0, 31, _bis,
                                  (jnp.int32(0), jnp.int32(_ONE_BITS)))
        gt = bits > lo
        eq = bits == lo
        quota = float(_K) - jnp.sum(gt.astype(jnp.float32))
        eqf = eq.astype(jnp.float32)
        within = jnp.dot(eqf, tri_incl, preferred_element_type=jnp.float32)
        rowtot = within[:, _C - 1:_C]
        rowpref = jnp.dot(tri_strict, rowtot,
                          preferred_element_type=jnp.float32)
        rank_incl = within + rowpref
        cand = jnp.logical_or(gt, jnp.logical_and(eq, rank_incl <= quota))
        SC[b][...] = jnp.where(cand, s, -1.0)

    out_ref[...] = jnp.zeros((_B, _MAX_OUT, _C), jnp.float32)

    fif = (jax.lax.broadcasted_iota(jnp.int32, (_R, _C), 0) * _C
           + jax.lax.broadcasted_iota(jnp.int32, (_R, _C), 1)
           ).astype(jnp.float32)
    lane_i = jax.lax.broadcasted_iota(jnp.int32, (1, _C), 1)
    lane = lane_i.astype(jnp.float32)

    def _round_one(b, cnt):
        scur = SC[b][...]
        m = jnp.max(scur, keepdims=True)
        f1 = jnp.min(jnp.where(scur == m, fif, 3.0e7), keepdims=True)
        r = f1[0, 0].astype(jnp.int32) // _C
        cv = f1 - jnp.floor(f1 * (1.0 / _C)) * float(_C)
        onehot = (lane == cv).astype(jnp.float32)
        by1 = jnp.sum(Y1[b][pl.ds(r, 1), :] * onehot, keepdims=True)
        bx1 = jnp.sum(X1[b][pl.ds(r, 1), :] * onehot, keepdims=True)
        by2 = jnp.sum(Y2[b][pl.ds(r, 1), :] * onehot, keepdims=True)
        bx2 = jnp.sum(X2[b][pl.ds(r, 1), :] * onehot, keepdims=True)
        yy1 = jnp.maximum(Y1[b][...], by1)
        xx1 = jnp.maximum(X1[b][...], bx1)
        yy2 = jnp.minimum(Y2[b][...], by2)
        xx2 = jnp.minimum(X2[b][...], bx2)
        inter = jnp.maximum(yy2 - yy1, 0.0) * jnp.maximum(xx2 - xx1, 0.0)
        union = areas[b] + (by2 - by1) * (bx2 - bx1) - inter
        iou = jnp.where(union > 0.0, inter / union, 0.0)
        kill = jnp.logical_or(iou > _TH, fif == f1)
        SC[b][...] = jnp.where(kill, -1.0, scur)
        pv = (m > -0.5).astype(jnp.float32)
        val = jnp.where(lane_i == 0, by1,
              jnp.where(lane_i == 1, bx1,
              jnp.where(lane_i == 2, by2,
              jnp.where(lane_i == 3, bx2, 0.0)))) * pv
        out_ref[b, pl.ds(cnt, 1), :] = val
        return cnt + (m[0, 0] > -0.5).astype(jnp.int32)

    def _round(_, cnts):
        return tuple(_round_one(b, cnts[b]) for b in range(_B))

    jax.lax.fori_loop(0, _MAX_OUT, _round, (jnp.int32(0),) * _B)


@jax.jit
def kernel(rpn_scores, rpn_bbox_delta, anchors):
    pad = _R * _C - _N

    def _planes(x3, pad_val):
        xp = jnp.pad(x3, ((0, 0), (0, pad), (0, 0)), constant_values=pad_val)
        return [xp[:, :, k].reshape(_B, _R, _C) for k in range(4)]

    scores = jnp.pad(rpn_scores[:, :, 1], ((0, 0), (0, pad)),
                     constant_values=-1.0).reshape(_B, _R, _C)
    d0, d1, d2, d3 = _planes(rpn_bbox_delta, 0.0)
    ap = jnp.pad(anchors, ((0, pad), (0, 0)))
    a0, a1, a2, a3 = [ap[:, k].reshape(_R, _C) for k in range(4)]

    out = pl.pallas_call(
        _nms_body,
        out_shape=jax.ShapeDtypeStruct((_B, _MAX_OUT, _C), jnp.float32),
        scratch_shapes=[pltpu.VMEM((_R, _C), jnp.float32)] * 10,
    )(scores, d0, d1, d2, d3, a0, a1, a2, a3)
    return out[:, :, :4]


# speedup vs baseline: 1.0155x; 1.0155x over previous
import jax
import jax.numpy as jnp
from jax.experimental import pallas as pl
from jax.experimental.pallas import tpu as pltpu

_B = 2
_N = 20000
_R = 160
_C = 128
_K = 6000
_MAX_OUT = 1000
_TH = 0.7
_STD = (0.1, 0.1, 0.2, 0.2)
_ONE_BITS = 0x3F800000


def _nms_body(s_ref, d0_ref, d1_ref, d2_ref, d3_ref,
              a0_ref, a1_ref, a2_ref, a3_ref, out_ref, *scr):
    SC = [scr[0], scr[6]]
    Y1 = [scr[1], scr[7]]
    X1 = [scr[2], scr[8]]
    Y2 = [scr[3], scr[9]]
    X2 = [scr[4], scr[10]]
    AR = [scr[5], scr[11]]
    fif_ref = scr[12]

    ay1, ax1, ay2, ax2 = a0_ref[...], a1_ref[...], a2_ref[...], a3_ref[...]
    ah = ay2 - ay1
    aw = ax2 - ax1
    acy = ay1 + 0.5 * ah
    acx = ax1 + 0.5 * aw

    li = jax.lax.broadcasted_iota(jnp.int32, (_C, _C), 0)
    lj = jax.lax.broadcasted_iota(jnp.int32, (_C, _C), 1)
    tri_incl = (li <= lj).astype(jnp.float32)
    ri = jax.lax.broadcasted_iota(jnp.int32, (_R, _R), 0)
    rj = jax.lax.broadcasted_iota(jnp.int32, (_R, _R), 1)
    tri_strict = (rj < ri).astype(jnp.float32)

    fif_ref[...] = (jax.lax.broadcasted_iota(jnp.int32, (_R, _C), 0) * _C
                    + jax.lax.broadcasted_iota(jnp.int32, (_R, _C), 1)
                    ).astype(jnp.float32)

    for b in range(_B):
        s = s_ref[b]
        cy = acy + (d0_ref[b] * _STD[0]) * ah
        cx = acx + (d1_ref[b] * _STD[1]) * aw
        h = ah * jnp.exp(d2_ref[b] * _STD[2])
        w = aw * jnp.exp(d3_ref[b] * _STD[3])
        y1 = jnp.clip(cy - 0.5 * h, 0.0, 1.0)
        x1 = jnp.clip(cx - 0.5 * w, 0.0, 1.0)
        y2 = jnp.clip((cy - 0.5 * h) + h, 0.0, 1.0)
        x2 = jnp.clip((cx - 0.5 * w) + w, 0.0, 1.0)
        Y1[b][...] = y1
        X1[b][...] = x1
        Y2[b][...] = y2
        X2[b][...] = x2
        AR[b][...] = (y2 - y1) * (x2 - x1)

        bits = jax.lax.bitcast_convert_type(s, jnp.int32)

        def _bis(_, lh):
            lo, hi = lh
            mid = (lo + hi) // 2
            cge = jnp.sum((bits >= mid).astype(jnp.float32))
            take = cge >= float(_K)
            return (jnp.where(take, mid, lo), jnp.where(take, hi, mid))

        lo, _ = jax.lax.fori_loop(0, 31, _bis,
                                  (jnp.int32(0), jnp.int32(_ONE_BITS)))
        gt = bits > lo
        eq = bits == lo
        quota = float(_K) - jnp.sum(gt.astype(jnp.float32))
        eqf = eq.astype(jnp.float32)
        within = jnp.dot(eqf, tri_incl, preferred_element_type=jnp.float32)
        rowtot = within[:, _C - 1:_C]
        rowpref = jnp.dot(tri_strict, rowtot,
                          preferred_element_type=jnp.float32)
        rank_incl = within + rowpref
        cand = jnp.logical_or(gt, jnp.logical_and(eq, rank_incl <= quota))
        SC[b][...] = jnp.where(cand, s, -1.0)

    lane_i = jax.lax.broadcasted_iota(jnp.int32, (1, _C), 1)
    lane = lane_i.astype(jnp.float32)

    def _round_one(b, i):
        scur = SC[b][...]
        m = jnp.max(scur, keepdims=True)
        f1 = jnp.min(jnp.where(scur == m, fif_ref[...], 3.0e7), keepdims=True)
        r = f1[0, 0].astype(jnp.int32) // _C
        cv = f1 - jnp.floor(f1 * (1.0 / _C)) * float(_C)
        onehot = (lane == cv).astype(jnp.float32)
        by1 = jnp.sum(Y1[b][pl.ds(r, 1), :] * onehot, keepdims=True)
        bx1 = jnp.sum(X1[b][pl.ds(r, 1), :] * onehot, keepdims=True)
        by2 = jnp.sum(Y2[b][pl.ds(r, 1), :] * onehot, keepdims=True)
        bx2 = jnp.sum(X2[b][pl.ds(r, 1), :] * onehot, keepdims=True)
        yy1 = jnp.maximum(Y1[b][...], by1)
        xx1 = jnp.maximum(X1[b][...], bx1)
        yy2 = jnp.minimum(Y2[b][...], by2)
        xx2 = jnp.minimum(X2[b][...], bx2)
        inter = jnp.maximum(yy2 - yy1, 0.0) * jnp.maximum(xx2 - xx1, 0.0)
        union = AR[b][...] + (by2 - by1) * (bx2 - bx1) - inter
        iou = jnp.where(union > 0.0, inter / union, 0.0)
        SC[b][...] = jnp.where(iou > _TH, -1.0, scur)
        SC[b][pl.ds(r, 1), :] = jnp.where(
            onehot > 0.0, -1.0, SC[b][pl.ds(r, 1), :])
        pv = (m > -0.5).astype(jnp.float32)
        val = jnp.where(lane_i == 0, by1,
              jnp.where(lane_i == 1, bx1,
              jnp.where(lane_i == 2, by2,
              jnp.where(lane_i == 3, bx2, 0.0)))) * pv
        out_ref[b, pl.ds(i, 1), :] = val

    def _round(i, carry):
        for b in range(_B):
            _round_one(b, i)
        return carry

    jax.lax.fori_loop(0, _MAX_OUT, _round, jnp.int32(0))


@jax.jit
def kernel(rpn_scores, rpn_bbox_delta, anchors):
    pad = _R * _C - _N

    def _planes(x3, pad_val):
        xp = jnp.pad(x3, ((0, 0), (0, pad), (0, 0)), constant_values=pad_val)
        return [xp[:, :, k].reshape(_B, _R, _C) for k in range(4)]

    scores = jnp.pad(rpn_scores[:, :, 1], ((0, 0), (0, pad)),
                     constant_values=-1.0).reshape(_B, _R, _C)
    d0, d1, d2, d3 = _planes(rpn_bbox_delta, 0.0)
    ap = jnp.pad(anchors, ((0, pad), (0, 0)))
    a0, a1, a2, a3 = [ap[:, k].reshape(_R, _C) for k in range(4)]

    out = pl.pallas_call(
        _nms_body,
        out_shape=jax.ShapeDtypeStruct((_B, _MAX_OUT, _C), jnp.float32),
        scratch_shapes=[pltpu.VMEM((_R, _C), jnp.float32)] * 13,
    )(scores, d0, d1, d2, d3, a0, a1, a2, a3)
    return out[:, :, :4]
